# baseline (speedup 1.0000x reference)
import functools

import jax
import jax.numpy as jnp
from jax import lax
from jax.experimental import pallas as pl
from jax.experimental.pallas import tpu as pltpu
from jax.experimental.pallas import tpu_sc as plsc

SEQ = 50
F = 32
E = 16


def _sc_gather_flat(t1d, station):
    info = plsc.get_sparse_core_info()
    nc, ns = info.num_cores, info.num_subcores
    nw = nc * ns
    B = station.shape[0]
    b_per_w = B // nw
    w_vals = b_per_w * E
    V = t1d.shape[0] // E
    mesh = plsc.VectorSubcoreMesh(core_axis_name="c", subcore_axis_name="s")

    @functools.partial(
        pl.kernel,
        mesh=mesh,
        compiler_params=pltpu.CompilerParams(
            use_tc_tiling_on_sc=False, needs_layout_passes=False
        ),
        out_type=jax.ShapeDtypeStruct((nw, 8, 128), jnp.float32),
        scratch_types=[
            pltpu.VMEM((b_per_w + E,), jnp.int32),
            pltpu.VMEM((w_vals,), jnp.int32),
            pltpu.VMEM((w_vals,), jnp.float32),
            pltpu.VMEM((8, 128), jnp.float32),
            pltpu.SemaphoreType.DMA,
        ],
    )
    def k(t_hbm, idx_hbm, out_hbm, idx_v, fidx_v, vals_v, flat_v, sem):
        wid = lax.axis_index("s") * nc + lax.axis_index("c")
        base = wid * b_per_w
        pltpu.sync_copy(idx_hbm.at[pl.ds(base, b_per_w)], idx_v.at[pl.ds(E, b_per_w)])
        cols = lax.iota(jnp.int32, E) * V
        copies = []
        for g in range(b_per_w):
            sg = plsc.load_gather(idx_v, [jnp.full((E,), E + g, jnp.int32)])
            copies.append(
                pltpu.async_copy(
                    t_hbm.at[sg + cols], vals_v.at[pl.ds(E * g, E)], sem
                )
            )
        for cp in copies:
            cp.wait()
        for g in range(b_per_w):
            flat_v[g // 8, pl.ds(E * (g % 8), E)] = vals_v[pl.ds(E * g, E)]
        pltpu.sync_copy(flat_v, out_hbm.at[wid])

    return k(t1d, station)


def _tc_concat(xt, e3):
    N = xt.shape[2]
    NB = 512
    grid = (N // NB,)

    def body(x_ref, e_ref, o_ref):
        o_ref[0:F, :, :] = jnp.swapaxes(x_ref[...], 0, 1)
        for c in range(NB // 128):
            ev = e_ref[0, c : c + 1, :].reshape(1, 1, 128)
            o_ref[F : F + 1, :, pl.ds(128 * c, 128)] = jnp.broadcast_to(
                ev, (1, SEQ, 128)
            )

    return pl.pallas_call(
        body,
        grid=grid,
        in_specs=[
            pl.BlockSpec((SEQ, F, NB), lambda j: (0, 0, j)),
            pl.BlockSpec((1, 8, 128), lambda j: (j, 0, 0)),
        ],
        out_specs=pl.BlockSpec((F + 1, SEQ, NB), lambda j: (0, 0, j)),
        out_shape=jax.ShapeDtypeStruct((F + 1, SEQ, N), jnp.float32),
    )(xt, e3)


def kernel(x, station, table):
    t1d = jnp.reshape(jnp.transpose(table, (1, 0)), (-1,))
    e3 = _sc_gather_flat(t1d, station)
    xt = jnp.transpose(x, (1, 2, 0))
    out_t = _tc_concat(xt, e3)
    return jnp.transpose(out_t, (2, 1, 0))


# speedup vs baseline: 3.8414x; 3.8414x over previous
import functools

import jax
import jax.numpy as jnp
from jax import lax
from jax.experimental import pallas as pl
from jax.experimental.pallas import tpu as pltpu
from jax.experimental.pallas import tpu_sc as plsc

SEQ = 50
F = 32
E = 16


def _tc_detile(tt):
    V = tt.shape[1]
    NB = 16384
    grid = ((V + NB - 1) // NB,)

    def body(t_ref, o_ref):
        x1 = t_ref[...].transpose(1, 0)
        xr = x1.reshape(NB // 8, 8, E)
        for w in range(8):
            o_ref[:, pl.ds(E * w, E)] = xr[:, w, :]

    return pl.pallas_call(
        body,
        grid=grid,
        in_specs=[pl.BlockSpec((E, NB), lambda j: (0, j))],
        out_specs=pl.BlockSpec((NB // 8, 128), lambda j: (j, 0)),
        out_shape=jax.ShapeDtypeStruct((V * E // 128, 128), jnp.float32),
    )(tt)


def _sc_gather_flat(t1d, station):
    info = plsc.get_sparse_core_info()
    nc, ns = info.num_cores, info.num_subcores
    nw = nc * ns
    B = station.shape[0]
    b_per_w = B // nw
    w_vals = b_per_w * E
    mesh = plsc.VectorSubcoreMesh(core_axis_name="c", subcore_axis_name="s")

    @functools.partial(
        pl.kernel,
        mesh=mesh,
        compiler_params=pltpu.CompilerParams(
            use_tc_tiling_on_sc=False, needs_layout_passes=False
        ),
        out_type=jax.ShapeDtypeStruct((nw, 8, 128), jnp.float32),
        scratch_types=[
            pltpu.VMEM((b_per_w + E,), jnp.int32),
            pltpu.VMEM((w_vals,), jnp.int32),
            pltpu.VMEM((w_vals,), jnp.float32),
            pltpu.VMEM((8, 128), jnp.float32),
            pltpu.SemaphoreType.DMA,
        ],
    )
    def k(t_hbm, idx_hbm, out_hbm, idx_v, fidx_v, vals_v, flat_v, sem):
        wid = lax.axis_index("s") * nc + lax.axis_index("c")
        base = wid * b_per_w
        pltpu.sync_copy(idx_hbm.at[pl.ds(base, b_per_w)], idx_v.at[pl.ds(E, b_per_w)])
        cols = lax.iota(jnp.int32, E)
        copies = []
        for g in range(b_per_w):
            sg = plsc.load_gather(idx_v, [jnp.full((E,), E + g, jnp.int32)])
            copies.append(
                pltpu.async_copy(
                    t_hbm.at[sg * E + cols], vals_v.at[pl.ds(E * g, E)], sem
                )
            )
        for cp in copies:
            cp.wait()
        for g in range(b_per_w):
            flat_v[g // 8, pl.ds(E * (g % 8), E)] = vals_v[pl.ds(E * g, E)]
        pltpu.sync_copy(flat_v, out_hbm.at[wid])

    return k(t1d, station)


def _tc_concat(xt, e3):
    N = xt.shape[2]
    NB = 512
    grid = (N // NB,)

    def body(x_ref, e_ref, o_ref):
        o_ref[0:F, :, :] = jnp.swapaxes(x_ref[...], 0, 1)
        for c in range(NB // 128):
            ev = e_ref[0, c : c + 1, :].reshape(1, 1, 128)
            o_ref[F : F + 1, :, pl.ds(128 * c, 128)] = jnp.broadcast_to(
                ev, (1, SEQ, 128)
            )

    return pl.pallas_call(
        body,
        grid=grid,
        in_specs=[
            pl.BlockSpec((SEQ, F, NB), lambda j: (0, 0, j)),
            pl.BlockSpec((1, 8, 128), lambda j: (j, 0, 0)),
        ],
        out_specs=pl.BlockSpec((F + 1, SEQ, NB), lambda j: (0, 0, j)),
        out_shape=jax.ShapeDtypeStruct((F + 1, SEQ, N), jnp.float32),
    )(xt, e3)


def kernel(x, station, table):
    tt = jnp.transpose(table, (1, 0))
    t1d = jnp.reshape(_tc_detile(tt), (-1,))
    e3 = _sc_gather_flat(t1d, station)
    xt = jnp.transpose(x, (1, 2, 0))
    out_t = _tc_concat(xt, e3)
    return jnp.transpose(out_t, (2, 1, 0))


# speedup vs baseline: 8.4145x; 2.1905x over previous
import functools

import jax
import jax.numpy as jnp
from jax import lax
from jax.experimental import pallas as pl
from jax.experimental.pallas import tpu as pltpu
from jax.experimental.pallas import tpu_sc as plsc

SEQ = 50
F = 32
E = 16


def _tc_detile(tt):
    V = tt.shape[1]
    NB = 16384
    NT = NB // 128
    grid = ((V + NB - 1) // NB,)
    rows = (V + 127) // 128

    def body(t_ref, o_ref):
        x3 = t_ref[...].reshape(E, NT, 128)
        o_ref[...] = x3.transpose(1, 0, 2)

    return pl.pallas_call(
        body,
        grid=grid,
        in_specs=[pl.BlockSpec((E, NB), lambda j: (0, j))],
        out_specs=pl.BlockSpec((NT, E, 128), lambda j: (j, 0, 0)),
        out_shape=jax.ShapeDtypeStruct((rows, E, 128), jnp.float32),
    )(tt)


def _sc_gather_flat(t1d, station):
    info = plsc.get_sparse_core_info()
    nc, ns = info.num_cores, info.num_subcores
    nw = nc * ns
    B = station.shape[0]
    b_per_w = B // nw
    w_vals = b_per_w * E
    mesh = plsc.VectorSubcoreMesh(core_axis_name="c", subcore_axis_name="s")

    @functools.partial(
        pl.kernel,
        mesh=mesh,
        compiler_params=pltpu.CompilerParams(
            use_tc_tiling_on_sc=False, needs_layout_passes=False
        ),
        out_type=jax.ShapeDtypeStruct((nw, 8, 128), jnp.float32),
        scratch_types=[
            pltpu.VMEM((b_per_w + E,), jnp.int32),
            pltpu.VMEM((w_vals,), jnp.int32),
            pltpu.VMEM((w_vals,), jnp.float32),
            pltpu.VMEM((8, 128), jnp.float32),
            pltpu.SemaphoreType.DMA,
        ],
    )
    def k(t_hbm, idx_hbm, out_hbm, idx_v, fidx_v, vals_v, flat_v, sem):
        wid = lax.axis_index("s") * nc + lax.axis_index("c")
        base = wid * b_per_w
        pltpu.sync_copy(idx_hbm.at[pl.ds(base, b_per_w)], idx_v.at[pl.ds(E, b_per_w)])
        cols = lax.iota(jnp.int32, E) * 128
        copies = []
        for g in range(b_per_w):
            sg = plsc.load_gather(idx_v, [jnp.full((E,), E + g, jnp.int32)])
            fidx = (sg // 128) * 2048 + cols + sg % 128
            copies.append(
                pltpu.async_copy(
                    t_hbm.at[fidx], vals_v.at[pl.ds(E * g, E)], sem
                )
            )
        for cp in copies:
            cp.wait()
        for g in range(b_per_w):
            flat_v[g // 8, pl.ds(E * (g % 8), E)] = vals_v[pl.ds(E * g, E)]
        pltpu.sync_copy(flat_v, out_hbm.at[wid])

    return k(t1d, station)


def _tc_concat(xt, e3):
    N = xt.shape[2]
    NB = 512
    grid = (N // NB,)

    def body(x_ref, e_ref, o_ref):
        o_ref[0:F, :, :] = jnp.swapaxes(x_ref[...], 0, 1)
        for c in range(NB // 128):
            ev = e_ref[0, c : c + 1, :].reshape(1, 1, 128)
            o_ref[F : F + 1, :, pl.ds(128 * c, 128)] = jnp.broadcast_to(
                ev, (1, SEQ, 128)
            )

    return pl.pallas_call(
        body,
        grid=grid,
        in_specs=[
            pl.BlockSpec((SEQ, F, NB), lambda j: (0, 0, j)),
            pl.BlockSpec((1, 8, 128), lambda j: (j, 0, 0)),
        ],
        out_specs=pl.BlockSpec((F + 1, SEQ, NB), lambda j: (0, 0, j)),
        out_shape=jax.ShapeDtypeStruct((F + 1, SEQ, N), jnp.float32),
    )(xt, e3)


def kernel(x, station, table):
    tt = jnp.transpose(table, (1, 0))
    t1d = jnp.reshape(_tc_detile(tt), (-1,))
    e3 = _sc_gather_flat(t1d, station)
    xt = jnp.transpose(x, (1, 2, 0))
    out_t = _tc_concat(xt, e3)
    return jnp.transpose(out_t, (2, 1, 0))
